# baseline (speedup 1.0000x reference)
import jax
import jax.numpy as jnp
from jax.experimental import pallas as pl
from jax.experimental.pallas import tpu as pltpu

_BR = 1000


def _copy_body(src_ref, dst_ref):
    dst_ref[...] = src_ref[...]


def kernel(embeddings):
    rows, dim = embeddings.shape
    v = embeddings.reshape(8, rows // 8, dim)
    grid = (rows // 8) // _BR
    out = pl.pallas_call(
        _copy_body,
        out_shape=jax.ShapeDtypeStruct(v.shape, v.dtype),
        grid=(grid,),
        in_specs=[pl.BlockSpec((8, _BR, dim), lambda i: (0, i, 0))],
        out_specs=pl.BlockSpec((8, _BR, dim), lambda i: (0, i, 0)),
    )(v)
    return out.reshape(rows, dim)
